# baseline (speedup 1.0000x reference)
import functools

import jax
import jax.numpy as jnp
from jax import lax
from jax.experimental import pallas as pl
from jax.experimental.pallas import tpu as pltpu
from jax.experimental.pallas import tpu_sc as plsc

VOCAB = 1000
B, T = 1024, 200
N = B * T

NC, NS = 2, 16
NW = NC * NS
R = N // NW
K = 40
NCHUNK = R // K
NBUF = 2

_mesh = plsc.VectorSubcoreMesh(core_axis_name="c", subcore_axis_name="s")


@functools.partial(
    pl.kernel,
    out_type=jax.ShapeDtypeStruct((N, VOCAB), jnp.float32),
    mesh=_mesh,
    scratch_types=[
        pltpu.VMEM((NCHUNK, K), jnp.int32),
        pltpu.VMEM((NBUF, K, VOCAB), jnp.float32),
        pltpu.SemaphoreType.DMA,
        pltpu.SemaphoreType.DMA,
    ],
    compiler_params=pltpu.CompilerParams(use_tc_tiling_on_sc=False),
)
def _gather_rows(idx_hbm, table_hbm, out_hbm, idx_v, rows_v, sem0, sem1):
    sems = (sem0, sem1)
    wid = lax.axis_index("s") * NC + lax.axis_index("c")
    base = wid * R

    pltpu.sync_copy(idx_hbm.at[wid], idx_v)

    for b in range(NBUF):
        pltpu.async_copy(table_hbm.at[idx_v.at[b]], rows_v.at[b], sems[b])

    def outer(g, carry):
        for b in range(NBUF):
            j = g * NBUF + b
            pltpu.make_async_copy(
                table_hbm.at[idx_v.at[j]], rows_v.at[b], sems[b]
            ).wait()
            pltpu.sync_copy(rows_v.at[b], out_hbm.at[pl.ds(base + j * K, K)])

            @pl.when(j + NBUF < NCHUNK)
            def _():
                pltpu.async_copy(
                    table_hbm.at[idx_v.at[j + NBUF]], rows_v.at[b], sems[b]
                )
        return carry

    lax.fori_loop(0, NCHUNK // NBUF, outer, 0)


def kernel(idx, table):
    idx_flat = idx.reshape(NW, NCHUNK, K).astype(jnp.int32)
    out = _gather_rows(idx_flat, table)
    return out.reshape(B, T, VOCAB)


# speedup vs baseline: 1.3172x; 1.3172x over previous
import functools

import jax
import jax.numpy as jnp
from jax import lax
from jax.experimental import pallas as pl
from jax.experimental.pallas import tpu as pltpu
from jax.experimental.pallas import tpu_sc as plsc

VOCAB = 1000
B, T = 1024, 200
N = B * T
VPAD = 1024

NC, NS = 2, 16
NW = NC * NS
R = N // NW
K = 40
NCHUNK = R // K
NBUF = 2

BB = 128
TB = 8

_mesh = plsc.VectorSubcoreMesh(core_axis_name="c", subcore_axis_name="s")


@functools.partial(
    pl.kernel,
    out_type=jax.ShapeDtypeStruct((N, VPAD), jnp.float32),
    mesh=_mesh,
    scratch_types=[
        pltpu.VMEM((R,), jnp.int32),
        pltpu.VMEM((NBUF, K, VPAD), jnp.float32),
        pltpu.SemaphoreType.DMA,
        pltpu.SemaphoreType.DMA,
    ],
)
def _gather_rows(idx_hbm, table_hbm, out_hbm, idx_v, rows_v, sem0, sem1):
    sems = (sem0, sem1)
    wid = lax.axis_index("s") * NC + lax.axis_index("c")
    base = wid * R

    pltpu.sync_copy(idx_hbm.at[pl.ds(base, R)], idx_v)

    def start(j, b):
        pltpu.async_copy(
            table_hbm.at[idx_v.at[pl.ds(j * K, K)]], rows_v.at[b], sems[b])

    for b in range(NBUF):
        start(b, b)

    def outer(g, carry):
        for b in range(NBUF):
            j = g * NBUF + b
            pltpu.make_async_copy(
                table_hbm.at[idx_v.at[pl.ds(j * K, K)]], rows_v.at[b], sems[b]
            ).wait()
            pltpu.sync_copy(rows_v.at[b], out_hbm.at[pl.ds(base + j * K, K)])

            @pl.when(j + NBUF < NCHUNK)
            def _():
                start(j + NBUF, b)
        return carry

    lax.fori_loop(0, NCHUNK // NBUF, outer, 0)


def _transpose_body(x_ref, eye_ref, o_ref):
    eye = eye_ref[...]
    for t in range(TB):
        xt = x_ref[:, t, :]
        y = lax.dot_general(
            xt, eye, (((0,), (0,)), ((), ())),
            preferred_element_type=jnp.float32,
            precision=lax.Precision.HIGHEST,
        )
        o_ref[t, :, :] = y[:VOCAB, :]


def _transpose(scratch3, eye):
    return pl.pallas_call(
        _transpose_body,
        grid=(T // TB, B // BB),
        in_specs=[
            pl.BlockSpec((BB, TB, VPAD), lambda tb, bb: (bb, tb, 0)),
            pl.BlockSpec((BB, BB), lambda tb, bb: (0, 0)),
        ],
        out_specs=pl.BlockSpec((TB, VOCAB, BB), lambda tb, bb: (tb, 0, bb)),
        out_shape=jax.ShapeDtypeStruct((T, VOCAB, B), jnp.float32),
    )(scratch3, eye)


def kernel(idx, table):
    idx_flat = idx.reshape(N).astype(jnp.int32)
    table_p = jnp.pad(table, ((0, 0), (0, VPAD - VOCAB)))
    scratch = _gather_rows(idx_flat, table_p)
    scratch3 = scratch.reshape(B, T, VPAD)
    eye = jnp.eye(BB, dtype=jnp.float32)
    out2 = _transpose(scratch3, eye)
    return jnp.transpose(out2, (2, 0, 1))
